# baseline (speedup 1.0000x reference)
import functools

import jax
import jax.numpy as jnp
from jax import lax
from jax.experimental import pallas as pl
from jax.experimental.pallas import tpu as pltpu
from jax.experimental.pallas import tpu_sc as plsc

N = 10000
E = 320000
NF = 128
NH = 128
NC = 64

CHUNK = 128
NTILES = 32
NCHUNKS = 79
E_PAD = NTILES * CHUNK * NCHUNKS
NCH0 = 78
NCH1 = 2 * NCHUNKS - NCH0
N_ACC = 10112
ZROWS = N_ACC // 16


def _make_sc_mp(F):
    mesh = plsc.VectorSubcoreMesh(core_axis_name="c", subcore_axis_name="s")

    @functools.partial(
        pl.kernel,
        out_type=jax.ShapeDtypeStruct((2, N_ACC, F), jnp.float32),
        mesh=mesh,
        scratch_types=[
            pltpu.VMEM((CHUNK,), jnp.int32),
            pltpu.VMEM((CHUNK,), jnp.int32),
            pltpu.VMEM((CHUNK,), jnp.int32),
            pltpu.VMEM((CHUNK,), jnp.int32),
            pltpu.VMEM((CHUNK, F), jnp.float32),
            pltpu.VMEM((CHUNK, F), jnp.float32),
            pltpu.SemaphoreType.DMA,
            pltpu.SemaphoreType.DMA,
            pltpu.VMEM_SHARED((N_ACC, F), jnp.float32),
        ],
    )
    def mp(z_hbm, srcg_hbm, dsts_hbm, zeros_hbm, out_hbm,
           src_a, dst_a, src_b, dst_b, rows_a, rows_b, sem_a, sem_b, acc):
        c = lax.axis_index("c")
        s = lax.axis_index("s")
        base0 = (s * (2 * NCHUNKS) + c * NCH0) * CHUNK
        nch = jnp.where(c == 0, NCH0, NCH1)
        pltpu.sync_copy(srcg_hbm.at[pl.ds(base0, CHUNK)], src_a)
        pltpu.sync_copy(dsts_hbm.at[pl.ds(base0, CHUNK)], dst_a)
        pltpu.async_copy(z_hbm.at[src_a], rows_a, sem_a)
        pltpu.sync_copy(zeros_hbm.at[pl.ds(s * ZROWS, ZROWS)],
                        acc.at[pl.ds(s * ZROWS, ZROWS)])
        plsc.subcore_barrier()

        def body(i, carry):
            j = 2 * i
            bb = pl.multiple_of(base0 + (j + 1) * CHUNK, CHUNK)
            pltpu.sync_copy(srcg_hbm.at[pl.ds(bb, CHUNK)], src_b)
            pltpu.sync_copy(dsts_hbm.at[pl.ds(bb, CHUNK)], dst_b)
            pltpu.async_copy(z_hbm.at[src_b], rows_b, sem_b)
            pltpu.make_async_copy(z_hbm.at[src_a], rows_a, sem_a).wait()
            pltpu.sync_copy(rows_a, acc.at[dst_a], add=True)
            ba = pl.multiple_of(base0 + lax.rem(j + 2, nch) * CHUNK, CHUNK)
            pltpu.sync_copy(srcg_hbm.at[pl.ds(ba, CHUNK)], src_a)
            pltpu.sync_copy(dsts_hbm.at[pl.ds(ba, CHUNK)], dst_a)
            pltpu.async_copy(z_hbm.at[src_a], rows_a, sem_a)
            pltpu.make_async_copy(z_hbm.at[src_b], rows_b, sem_b).wait()
            pltpu.sync_copy(rows_b, acc.at[dst_b], add=True)
            return carry

        lax.fori_loop(0, nch // 2, body, 0)
        pltpu.make_async_copy(z_hbm.at[src_a], rows_a, sem_a).wait()
        plsc.subcore_barrier()
        pltpu.sync_copy(acc.at[pl.ds(s * ZROWS, ZROWS)],
                        out_hbm.at[c].at[pl.ds(s * ZROWS, ZROWS)])

    return mp


_sc_mp_128 = _make_sc_mp(NH)


DEGW = 128


def _make_sc_deg():
    mesh = plsc.VectorSubcoreMesh(core_axis_name="c", subcore_axis_name="s")

    @functools.partial(
        pl.kernel,
        out_type=jax.ShapeDtypeStruct((2, N_ACC, DEGW), jnp.float32),
        mesh=mesh,
        scratch_types=[
            pltpu.VMEM((CHUNK,), jnp.int32),
            pltpu.VMEM((CHUNK,), jnp.int32),
            pltpu.VMEM((CHUNK, DEGW), jnp.float32),
            pltpu.SemaphoreType.DMA,
            pltpu.SemaphoreType.DMA,
            pltpu.VMEM_SHARED((N_ACC, DEGW), jnp.float32),
        ],
    )
    def deg(srcd_hbm, ones_hbm, zeros_hbm, out_hbm,
            src_a, src_b, ones_v, sem_a, sem_b, dacc):
        c = lax.axis_index("c")
        s = lax.axis_index("s")
        base0 = (s * (2 * NCHUNKS) + c * NCH0) * CHUNK
        nch = jnp.where(c == 0, NCH0, NCH1)
        pltpu.async_copy(srcd_hbm.at[pl.ds(base0, CHUNK)], src_a, sem_a)
        pltpu.sync_copy(zeros_hbm.at[pl.ds(s * ZROWS, ZROWS)],
                        dacc.at[pl.ds(s * ZROWS, ZROWS)])
        pltpu.sync_copy(ones_hbm, ones_v)
        plsc.subcore_barrier()

        def body(i, carry):
            j = 2 * i
            bb = pl.multiple_of(base0 + (j + 1) * CHUNK, CHUNK)
            pltpu.async_copy(srcd_hbm.at[pl.ds(bb, CHUNK)], src_b, sem_b)
            pltpu.make_async_copy(srcd_hbm.at[pl.ds(bb, CHUNK)], src_a,
                                  sem_a).wait()
            pltpu.sync_copy(ones_v, dacc.at[src_a], add=True)
            ba = pl.multiple_of(base0 + lax.rem(j + 2, nch) * CHUNK, CHUNK)
            pltpu.async_copy(srcd_hbm.at[pl.ds(ba, CHUNK)], src_a, sem_a)
            pltpu.make_async_copy(srcd_hbm.at[pl.ds(bb, CHUNK)], src_b,
                                  sem_b).wait()
            pltpu.sync_copy(ones_v, dacc.at[src_b], add=True)
            return carry

        lax.fori_loop(0, nch // 2, body, 0)
        pltpu.make_async_copy(srcd_hbm.at[pl.ds(base0, CHUNK)], src_a,
                              sem_a).wait()
        plsc.subcore_barrier()
        pltpu.sync_copy(dacc.at[pl.ds(s * ZROWS, ZROWS)],
                        out_hbm.at[c].at[pl.ds(s * ZROWS, ZROWS)])

    return deg


_sc_deg = _make_sc_deg()


BM = 1000


def _tc_pre_body(x_ref, w0_ref, w1_ref, b_ref, degp_ref,
                 a0_ref, z_ref, dinv_ref):
    deg = (degp_ref[0] + degp_ref[1])[:, 0:1]
    dinv = jnp.where(deg > 0, 1.0 / jnp.sqrt(jnp.maximum(deg, 1.0)), 0.0)
    x = x_ref[...]
    a0_ref[...] = jnp.dot(x, w0_ref[...],
                          preferred_element_type=jnp.float32,
                          precision=lax.Precision.HIGHEST) + b_ref[...]
    z_ref[...] = jnp.dot(dinv * x, w1_ref[...],
                         preferred_element_type=jnp.float32,
                         precision=lax.Precision.HIGHEST)
    dinv_ref[...] = dinv


def _tc_pre(x, w0, w1, b, degp):
    grid = (N // BM,)
    return pl.pallas_call(
        _tc_pre_body,
        grid=grid,
        in_specs=[
            pl.BlockSpec((BM, NF), lambda i: (i, 0)),
            pl.BlockSpec((NF, NH), lambda i: (0, 0)),
            pl.BlockSpec((NF, NH), lambda i: (0, 0)),
            pl.BlockSpec((1, NH), lambda i: (0, 0)),
            pl.BlockSpec((2, BM, DEGW), lambda i: (0, i, 0)),
        ],
        out_specs=[
            pl.BlockSpec((BM, NH), lambda i: (i, 0)),
            pl.BlockSpec((BM, NH), lambda i: (i, 0)),
            pl.BlockSpec((BM, 1), lambda i: (i, 0)),
        ],
        out_shape=[
            jax.ShapeDtypeStruct((N, NH), jnp.float32),
            jax.ShapeDtypeStruct((N, NH), jnp.float32),
            jax.ShapeDtypeStruct((N, 1), jnp.float32),
        ],
    )(x, w0, w1, b, degp)


def _tc_mid_body(a0p_ref, sp_ref, dinv_ref, w0_ref, w1_ref, b_ref,
                 a0_ref, z_ref):
    svec = sp_ref[0] + sp_ref[1]
    dinv = dinv_ref[...]
    h = jnp.maximum(a0p_ref[...] - dinv * svec, 0.0)
    a0_ref[...] = jnp.dot(h, w0_ref[...],
                          preferred_element_type=jnp.float32,
                          precision=lax.Precision.HIGHEST) + b_ref[...]
    z_ref[...] = jnp.dot(dinv * h, w1_ref[...],
                         preferred_element_type=jnp.float32,
                         precision=lax.Precision.HIGHEST)


def _tc_mid(a0p, sp, dinv, w0, w1, b):
    grid = (N // BM,)
    fin = a0p.shape[1]
    f0 = w0.shape[1]
    f1 = w1.shape[1]
    return pl.pallas_call(
        _tc_mid_body,
        grid=grid,
        in_specs=[
            pl.BlockSpec((BM, fin), lambda i: (i, 0)),
            pl.BlockSpec((2, BM, fin), lambda i: (0, i, 0)),
            pl.BlockSpec((BM, 1), lambda i: (i, 0)),
            pl.BlockSpec((fin, f0), lambda i: (0, 0)),
            pl.BlockSpec((fin, f1), lambda i: (0, 0)),
            pl.BlockSpec((1, f0), lambda i: (0, 0)),
        ],
        out_specs=[
            pl.BlockSpec((BM, f0), lambda i: (i, 0)),
            pl.BlockSpec((BM, f1), lambda i: (i, 0)),
        ],
        out_shape=[
            jax.ShapeDtypeStruct((N, f0), jnp.float32),
            jax.ShapeDtypeStruct((N, f1), jnp.float32),
        ],
    )(a0p, sp, dinv, w0, w1, b)


def _tc_post_body(a0p_ref, sp_ref, dinv_ref, out_ref):
    svec = (sp_ref[0] + sp_ref[1])[:, :NC]
    out_ref[...] = a0p_ref[...] - dinv_ref[...] * svec


def _tc_post(a0p, sp, dinv):
    grid = (N // BM,)
    f = a0p.shape[1]
    return pl.pallas_call(
        _tc_post_body,
        grid=grid,
        in_specs=[
            pl.BlockSpec((BM, f), lambda i: (i, 0)),
            pl.BlockSpec((2, BM, NH), lambda i: (0, i, 0)),
            pl.BlockSpec((BM, 1), lambda i: (i, 0)),
        ],
        out_specs=pl.BlockSpec((BM, f), lambda i: (i, 0)),
        out_shape=jax.ShapeDtypeStruct((N, f), jnp.float32),
    )(a0p, sp, dinv)


def kernel(x, adj, W0_1, W1_1, b1, W0_2, W1_2, b2, W0_3, W1_3, b3):
    src = adj[0].astype(jnp.int32)
    dst = adj[1].astype(jnp.int32)
    pad = E_PAD - E
    sink = N + (jnp.arange(pad, dtype=jnp.int32) % (N_ACC - N))
    srcg = jnp.concatenate([src, jnp.zeros((pad,), jnp.int32)])
    dsts = jnp.concatenate([dst, sink])
    srcd = jnp.concatenate([src, sink])
    ones_rows = jnp.ones((CHUNK, DEGW), jnp.float32)
    zeros128 = jnp.zeros((N_ACC, NH), jnp.float32)
    w1_3p = jnp.pad(W1_3, ((0, 0), (0, NH - NC)))

    degp = _sc_deg(srcd, ones_rows, zeros128)
    a0, z, dinv = _tc_pre(x, W0_1, W1_1, b1.reshape(1, -1), degp)
    s1 = _sc_mp_128(z, srcg, dsts, zeros128)
    a0, z = _tc_mid(a0, s1, dinv, W0_2, W1_2, b2.reshape(1, -1))
    s2 = _sc_mp_128(z, srcg, dsts, zeros128)
    a0, z = _tc_mid(a0, s2, dinv, W0_3, w1_3p, b3.reshape(1, -1))
    s3 = _sc_mp_128(z, srcg, dsts, zeros128)
    return _tc_post(a0, s3, dinv)


# speedup vs baseline: 1.0039x; 1.0039x over previous
import functools

import jax
import jax.numpy as jnp
from jax import lax
from jax.experimental import pallas as pl
from jax.experimental.pallas import tpu as pltpu
from jax.experimental.pallas import tpu_sc as plsc

N = 10000
E = 320000
NF = 128
NH = 128
NC = 64

CHUNK = 128
NTILES = 32
NCHUNKS = 79
E_PAD = NTILES * CHUNK * NCHUNKS
NCH0 = 78
NCH1 = 2 * NCHUNKS - NCH0
N_ACC = 10112
ZROWS = N_ACC // 16


def _make_sc_mp(F, untiled=False):
    mesh = plsc.VectorSubcoreMesh(core_axis_name="c", subcore_axis_name="s")
    params = (pltpu.CompilerParams(use_tc_tiling_on_sc=False)
              if untiled else None)

    @functools.partial(
        pl.kernel,
        out_type=jax.ShapeDtypeStruct((2, N_ACC, F), jnp.float32),
        mesh=mesh,
        compiler_params=params,
        scratch_types=[
            pltpu.VMEM((CHUNK,), jnp.int32),
            pltpu.VMEM((CHUNK,), jnp.int32),
            pltpu.VMEM((CHUNK,), jnp.int32),
            pltpu.VMEM((CHUNK,), jnp.int32),
            pltpu.VMEM((CHUNK, F), jnp.float32),
            pltpu.VMEM((CHUNK, F), jnp.float32),
            pltpu.SemaphoreType.DMA,
            pltpu.SemaphoreType.DMA,
            pltpu.VMEM_SHARED((N_ACC, F), jnp.float32),
        ],
    )
    def mp(z_hbm, srcg_hbm, dsts_hbm, zeros_hbm, out_hbm,
           src_a, dst_a, src_b, dst_b, rows_a, rows_b, sem_a, sem_b, acc):
        c = lax.axis_index("c")
        s = lax.axis_index("s")
        base0 = (s * (2 * NCHUNKS) + c * NCH0) * CHUNK
        nch = jnp.where(c == 0, NCH0, NCH1)
        pltpu.sync_copy(srcg_hbm.at[pl.ds(base0, CHUNK)], src_a)
        pltpu.sync_copy(dsts_hbm.at[pl.ds(base0, CHUNK)], dst_a)
        pltpu.async_copy(z_hbm.at[src_a], rows_a, sem_a)
        pltpu.sync_copy(zeros_hbm.at[pl.ds(s * ZROWS, ZROWS)],
                        acc.at[pl.ds(s * ZROWS, ZROWS)])
        plsc.subcore_barrier()

        def body(i, carry):
            j = 2 * i
            bb = pl.multiple_of(base0 + (j + 1) * CHUNK, CHUNK)
            pltpu.sync_copy(srcg_hbm.at[pl.ds(bb, CHUNK)], src_b)
            pltpu.sync_copy(dsts_hbm.at[pl.ds(bb, CHUNK)], dst_b)
            pltpu.async_copy(z_hbm.at[src_b], rows_b, sem_b)
            pltpu.make_async_copy(z_hbm.at[src_a], rows_a, sem_a).wait()
            pltpu.sync_copy(rows_a, acc.at[dst_a], add=True)
            ba = pl.multiple_of(base0 + lax.rem(j + 2, nch) * CHUNK, CHUNK)
            pltpu.sync_copy(srcg_hbm.at[pl.ds(ba, CHUNK)], src_a)
            pltpu.sync_copy(dsts_hbm.at[pl.ds(ba, CHUNK)], dst_a)
            pltpu.async_copy(z_hbm.at[src_a], rows_a, sem_a)
            pltpu.make_async_copy(z_hbm.at[src_b], rows_b, sem_b).wait()
            pltpu.sync_copy(rows_b, acc.at[dst_b], add=True)
            return carry

        lax.fori_loop(0, nch // 2, body, 0)
        pltpu.make_async_copy(z_hbm.at[src_a], rows_a, sem_a).wait()
        plsc.subcore_barrier()
        pltpu.sync_copy(acc.at[pl.ds(s * ZROWS, ZROWS)],
                        out_hbm.at[c].at[pl.ds(s * ZROWS, ZROWS)])

    return mp


_sc_mp_128 = _make_sc_mp(NH)
_sc_mp_64 = _make_sc_mp(NC, untiled=True)


DEGW = 128


def _make_sc_deg():
    mesh = plsc.VectorSubcoreMesh(core_axis_name="c", subcore_axis_name="s")

    @functools.partial(
        pl.kernel,
        out_type=jax.ShapeDtypeStruct((2, N_ACC, DEGW), jnp.float32),
        mesh=mesh,
        scratch_types=[
            pltpu.VMEM((CHUNK,), jnp.int32),
            pltpu.VMEM((CHUNK,), jnp.int32),
            pltpu.VMEM((CHUNK, DEGW), jnp.float32),
            pltpu.SemaphoreType.DMA,
            pltpu.SemaphoreType.DMA,
            pltpu.VMEM_SHARED((N_ACC, DEGW), jnp.float32),
        ],
    )
    def deg(srcd_hbm, ones_hbm, zeros_hbm, out_hbm,
            src_a, src_b, ones_v, sem_a, sem_b, dacc):
        c = lax.axis_index("c")
        s = lax.axis_index("s")
        base0 = (s * (2 * NCHUNKS) + c * NCH0) * CHUNK
        nch = jnp.where(c == 0, NCH0, NCH1)
        pltpu.async_copy(srcd_hbm.at[pl.ds(base0, CHUNK)], src_a, sem_a)
        pltpu.sync_copy(zeros_hbm.at[pl.ds(s * ZROWS, ZROWS)],
                        dacc.at[pl.ds(s * ZROWS, ZROWS)])
        pltpu.sync_copy(ones_hbm, ones_v)
        plsc.subcore_barrier()

        def body(i, carry):
            j = 2 * i
            bb = pl.multiple_of(base0 + (j + 1) * CHUNK, CHUNK)
            pltpu.async_copy(srcd_hbm.at[pl.ds(bb, CHUNK)], src_b, sem_b)
            pltpu.make_async_copy(srcd_hbm.at[pl.ds(bb, CHUNK)], src_a,
                                  sem_a).wait()
            pltpu.sync_copy(ones_v, dacc.at[src_a], add=True)
            ba = pl.multiple_of(base0 + lax.rem(j + 2, nch) * CHUNK, CHUNK)
            pltpu.async_copy(srcd_hbm.at[pl.ds(ba, CHUNK)], src_a, sem_a)
            pltpu.make_async_copy(srcd_hbm.at[pl.ds(bb, CHUNK)], src_b,
                                  sem_b).wait()
            pltpu.sync_copy(ones_v, dacc.at[src_b], add=True)
            return carry

        lax.fori_loop(0, nch // 2, body, 0)
        pltpu.make_async_copy(srcd_hbm.at[pl.ds(base0, CHUNK)], src_a,
                              sem_a).wait()
        plsc.subcore_barrier()
        pltpu.sync_copy(dacc.at[pl.ds(s * ZROWS, ZROWS)],
                        out_hbm.at[c].at[pl.ds(s * ZROWS, ZROWS)])

    return deg


_sc_deg = _make_sc_deg()


BM = 1000


def _tc_pre_body(x_ref, w0_ref, w1_ref, b_ref, degp_ref,
                 a0_ref, z_ref, dinv_ref):
    deg = (degp_ref[0] + degp_ref[1])[:, 0:1]
    dinv = jnp.where(deg > 0, 1.0 / jnp.sqrt(jnp.maximum(deg, 1.0)), 0.0)
    x = x_ref[...]
    a0_ref[...] = jnp.dot(x, w0_ref[...],
                          preferred_element_type=jnp.float32,
                          precision=lax.Precision.HIGHEST) + b_ref[...]
    z_ref[...] = jnp.dot(dinv * x, w1_ref[...],
                         preferred_element_type=jnp.float32,
                         precision=lax.Precision.HIGHEST)
    dinv_ref[...] = dinv


def _tc_pre(x, w0, w1, b, degp):
    grid = (N // BM,)
    return pl.pallas_call(
        _tc_pre_body,
        grid=grid,
        in_specs=[
            pl.BlockSpec((BM, NF), lambda i: (i, 0)),
            pl.BlockSpec((NF, NH), lambda i: (0, 0)),
            pl.BlockSpec((NF, NH), lambda i: (0, 0)),
            pl.BlockSpec((1, NH), lambda i: (0, 0)),
            pl.BlockSpec((2, BM, DEGW), lambda i: (0, i, 0)),
        ],
        out_specs=[
            pl.BlockSpec((BM, NH), lambda i: (i, 0)),
            pl.BlockSpec((BM, NH), lambda i: (i, 0)),
            pl.BlockSpec((BM, 1), lambda i: (i, 0)),
        ],
        out_shape=[
            jax.ShapeDtypeStruct((N, NH), jnp.float32),
            jax.ShapeDtypeStruct((N, NH), jnp.float32),
            jax.ShapeDtypeStruct((N, 1), jnp.float32),
        ],
    )(x, w0, w1, b, degp)


def _tc_mid_body(a0p_ref, sp_ref, dinv_ref, w0_ref, w1_ref, b_ref,
                 a0_ref, z_ref):
    svec = sp_ref[0] + sp_ref[1]
    dinv = dinv_ref[...]
    h = jnp.maximum(a0p_ref[...] - dinv * svec, 0.0)
    a0_ref[...] = jnp.dot(h, w0_ref[...],
                          preferred_element_type=jnp.float32,
                          precision=lax.Precision.HIGHEST) + b_ref[...]
    z_ref[...] = jnp.dot(dinv * h, w1_ref[...],
                         preferred_element_type=jnp.float32,
                         precision=lax.Precision.HIGHEST)


def _tc_mid(a0p, sp, dinv, w0, w1, b):
    grid = (N // BM,)
    fin = a0p.shape[1]
    f0 = w0.shape[1]
    f1 = w1.shape[1]
    return pl.pallas_call(
        _tc_mid_body,
        grid=grid,
        in_specs=[
            pl.BlockSpec((BM, fin), lambda i: (i, 0)),
            pl.BlockSpec((2, BM, fin), lambda i: (0, i, 0)),
            pl.BlockSpec((BM, 1), lambda i: (i, 0)),
            pl.BlockSpec((fin, f0), lambda i: (0, 0)),
            pl.BlockSpec((fin, f1), lambda i: (0, 0)),
            pl.BlockSpec((1, f0), lambda i: (0, 0)),
        ],
        out_specs=[
            pl.BlockSpec((BM, f0), lambda i: (i, 0)),
            pl.BlockSpec((BM, f1), lambda i: (i, 0)),
        ],
        out_shape=[
            jax.ShapeDtypeStruct((N, f0), jnp.float32),
            jax.ShapeDtypeStruct((N, f1), jnp.float32),
        ],
    )(a0p, sp, dinv, w0, w1, b)


def _tc_post_body(a0p_ref, sp_ref, dinv_ref, out_ref):
    out_ref[...] = a0p_ref[...] - dinv_ref[...] * (sp_ref[0] + sp_ref[1])


def _tc_post(a0p, sp, dinv):
    grid = (N // BM,)
    f = a0p.shape[1]
    return pl.pallas_call(
        _tc_post_body,
        grid=grid,
        in_specs=[
            pl.BlockSpec((BM, f), lambda i: (i, 0)),
            pl.BlockSpec((2, BM, NC), lambda i: (0, i, 0)),
            pl.BlockSpec((BM, 1), lambda i: (i, 0)),
        ],
        out_specs=pl.BlockSpec((BM, f), lambda i: (i, 0)),
        out_shape=jax.ShapeDtypeStruct((N, f), jnp.float32),
    )(a0p, sp, dinv)


def kernel(x, adj, W0_1, W1_1, b1, W0_2, W1_2, b2, W0_3, W1_3, b3):
    src = adj[0].astype(jnp.int32)
    dst = adj[1].astype(jnp.int32)
    pad = E_PAD - E
    sink = N + (jnp.arange(pad, dtype=jnp.int32) % (N_ACC - N))
    srcg = jnp.concatenate([src, jnp.zeros((pad,), jnp.int32)])
    dsts = jnp.concatenate([dst, sink])
    srcd = jnp.concatenate([src, sink])
    ones_rows = jnp.ones((CHUNK, DEGW), jnp.float32)
    zeros128 = jnp.zeros((N_ACC, NH), jnp.float32)
    zeros64 = jnp.zeros((N_ACC, NC), jnp.float32)

    degp = _sc_deg(srcd, ones_rows, zeros128)
    a0, z, dinv = _tc_pre(x, W0_1, W1_1, b1.reshape(1, -1), degp)
    s1 = _sc_mp_128(z, srcg, dsts, zeros128)
    a0, z = _tc_mid(a0, s1, dinv, W0_2, W1_2, b2.reshape(1, -1))
    s2 = _sc_mp_128(z, srcg, dsts, zeros128)
    a0, z = _tc_mid(a0, s2, dinv, W0_3, W1_3, b3.reshape(1, -1))
    s3 = _sc_mp_64(z, srcg, dsts, zeros64)
    return _tc_post(a0, s3, dinv)
